# Initial kernel scaffold; baseline (speedup 1.0000x reference)
import functools

import jax
import jax.numpy as jnp
from jax import lax
from jax.experimental import pallas as pl
from jax.experimental.pallas import tpu as pltpu
from jax.experimental.pallas import tpu_sc as plsc

N = 10000
E = 320000
H = 128
EIN = 16
DEPTH = 5
G = 64
HID = 64
BN_EPS = 1e-5

NW = 32
EPAD = 327680
EW = EPAD // NW
IDXROWS = EPAD // 128
IRW = EW // 128
CA = 512
CC = 256
NP = N + 16
NPR = NP // 16
NOR = N // 16

_mesh = plsc.VectorSubcoreMesh(core_axis_name="c", subcore_axis_name="s")


def _sc_pre_body(p1_hbm, p2_hbm, srcm_hbm, dstm_hbm, pre_hbm,
                 idx_s, idx_d, p1_v, p2_v, sem1, sem2):
    cid = lax.axis_index("c")
    sid = lax.axis_index("s")
    wid = sid * 2 + cid

    def chunk(k, carry):
        r0 = wid * IRW + k * (CA // 128)
        b0 = wid * EW + k * CA
        pltpu.sync_copy(srcm_hbm.at[pl.ds(r0, CA // 128)], idx_s)
        pltpu.sync_copy(dstm_hbm.at[pl.ds(r0, CA // 128)], idx_d)
        cps = []
        for j in range(CA // 128):
            cps.append(pltpu.async_copy(
                p1_hbm.at[idx_s.at[j]], p1_v.at[pl.ds(j * 128, 128)], sem1))
            cps.append(pltpu.async_copy(
                p2_hbm.at[idx_d.at[j]], p2_v.at[pl.ds(j * 128, 128)], sem2))
        for c in cps:
            c.wait()

        def addrow(r, carry2):
            for jj in range(HID // 16):
                s = pl.ds(jj * 16, 16)
                p1_v[r, s] = p1_v[r, s] + p2_v[r, s]
            return carry2
        lax.fori_loop(0, CA, addrow, 0)
        pltpu.sync_copy(p1_v, pre_hbm.at[pl.ds(b0, CA)])
        return carry
    lax.fori_loop(0, EW // CA, chunk, 0)


_sc_pre = functools.partial(
    pl.kernel,
    out_type=jax.ShapeDtypeStruct((EPAD, HID), jnp.float32),
    mesh=_mesh,
    scratch_types=[
        pltpu.VMEM((CA // 128, 128), jnp.int32),
        pltpu.VMEM((CA // 128, 128), jnp.int32),
        pltpu.VMEM((CA, HID), jnp.float32),
        pltpu.VMEM((CA, HID), jnp.float32),
        pltpu.SemaphoreType.DMA,
        pltpu.SemaphoreType.DMA,
    ],
)(_sc_pre_body)


def _sc_aggr_body(x_hbm, e_hbm, srcm_hbm, dstm_hbm, zeros_hbm, aggr_hbm,
                  idx_s, idx_d, xs_v, e_v, aggr_sh, sem1, sem2):
    cid = lax.axis_index("c")
    sid = lax.axis_index("s")
    wid = sid * 2 + cid

    pltpu.sync_copy(zeros_hbm.at[pl.ds(sid * NPR, NPR)],
                    aggr_sh.at[pl.ds(sid * NPR, NPR)])
    plsc.subcore_barrier()

    def chunk(k, carry):
        r0 = wid * IRW + k * (CC // 128)
        b0 = wid * EW + k * CC
        pltpu.sync_copy(srcm_hbm.at[pl.ds(r0, CC // 128)], idx_s)
        pltpu.sync_copy(dstm_hbm.at[pl.ds(r0, CC // 128)], idx_d)
        cps = [pltpu.async_copy(e_hbm.at[pl.ds(b0, CC)], e_v, sem2)]
        for j in range(CC // 128):
            cps.append(pltpu.async_copy(
                x_hbm.at[idx_s.at[j]], xs_v.at[pl.ds(j * 128, 128)], sem1))
        for c in cps:
            c.wait()

        def msgrow(r, carry2):
            for jj in range(H // 16):
                s = pl.ds(jj * 16, 16)
                xs_v[r, s] = jnp.maximum(xs_v[r, s] + e_v[r, s], 0.0)
            return carry2
        lax.fori_loop(0, CC, msgrow, 0)
        for j in range(CC // 128):
            pltpu.sync_copy(xs_v.at[pl.ds(j * 128, 128)],
                            aggr_sh.at[idx_d.at[j]], add=True)
        return carry
    lax.fori_loop(0, EW // CC, chunk, 0)

    plsc.subcore_barrier()
    pltpu.sync_copy(aggr_sh.at[pl.ds(sid * NOR, NOR)],
                    aggr_hbm.at[cid, pl.ds(sid * NOR, NOR)])


_sc_aggr = functools.partial(
    pl.kernel,
    out_type=jax.ShapeDtypeStruct((2, N, H), jnp.float32),
    mesh=_mesh,
    scratch_types=[
        pltpu.VMEM((CC // 128, 128), jnp.int32),
        pltpu.VMEM((CC // 128, 128), jnp.int32),
        pltpu.VMEM((CC, H), jnp.float32),
        pltpu.VMEM((CC, H), jnp.float32),
        pltpu.VMEM_SHARED((NP, H), jnp.float32),
        pltpu.SemaphoreType.DMA,
        pltpu.SemaphoreType.DMA,
    ],
)(_sc_aggr_body)


EB = 2048


def _dot(a, b):
    return jnp.dot(a, b, preferred_element_type=jnp.float32)


def _tc_edge_body(pre_ref, e_ref, w1c_ref, w2_ref, b2_ref, eo_ref):
    e = e_ref[...]
    h = jnp.maximum(pre_ref[...] + _dot(e, w1c_ref[...]), 0.0)
    eo_ref[...] = e + _dot(h, w2_ref[...]) + b2_ref[...]


_tc_edge = pl.pallas_call(
    _tc_edge_body,
    grid=(EPAD // EB,),
    in_specs=[
        pl.BlockSpec((EB, HID), lambda i: (i, 0)),
        pl.BlockSpec((EB, H), lambda i: (i, 0)),
        pl.BlockSpec((H, HID), lambda i: (0, 0)),
        pl.BlockSpec((HID, H), lambda i: (0, 0)),
        pl.BlockSpec((1, H), lambda i: (0, 0)),
    ],
    out_specs=pl.BlockSpec((EB, H), lambda i: (i, 0)),
    out_shape=jax.ShapeDtypeStruct((EPAD, H), jnp.float32),
)


def _tc_edge0_body(ea_ref, we_ref, be_ref, pre_ref, w1c_ref, w2_ref, b2_ref,
                   eo_ref):
    e = _dot(ea_ref[...], we_ref[...]) + be_ref[...]
    h = jnp.maximum(pre_ref[...] + _dot(e, w1c_ref[...]), 0.0)
    eo_ref[...] = e + _dot(h, w2_ref[...]) + b2_ref[...]


_tc_edge0 = pl.pallas_call(
    _tc_edge0_body,
    grid=(EPAD // EB,),
    in_specs=[
        pl.BlockSpec((EB, EIN), lambda i: (i, 0)),
        pl.BlockSpec((EIN, H), lambda i: (0, 0)),
        pl.BlockSpec((1, H), lambda i: (0, 0)),
        pl.BlockSpec((EB, HID), lambda i: (i, 0)),
        pl.BlockSpec((H, HID), lambda i: (0, 0)),
        pl.BlockSpec((HID, H), lambda i: (0, 0)),
        pl.BlockSpec((1, H), lambda i: (0, 0)),
    ],
    out_specs=pl.BlockSpec((EB, H), lambda i: (i, 0)),
    out_shape=jax.ShapeDtypeStruct((EPAD, H), jnp.float32),
)


def _node_update(x, a0, a1, w1, b1, w2, b2, gam, bet):
    t = x + a0 + a1
    u = _dot(jnp.maximum(_dot(t, w1) + b1, 0.0), w2) + b2
    mean = jnp.mean(u, axis=0, keepdims=True)
    var = jnp.mean((u - mean) * (u - mean), axis=0, keepdims=True)
    xb = (u - mean) * lax.rsqrt(var + BN_EPS) * gam + bet
    return x + jnp.maximum(xb, 0.0)


def _tc_node_body(x_ref, a0_ref, a1_ref, w1_ref, b1_ref, w2_ref, b2_ref,
                  gam_ref, bet_ref, nwa_ref, nwb_ref, nb_ref,
                  xo_ref, p1_ref, p2_ref):
    xn = _node_update(x_ref[...], a0_ref[...], a1_ref[...], w1_ref[...],
                      b1_ref[...], w2_ref[...], b2_ref[...], gam_ref[...],
                      bet_ref[...])
    xo_ref[...] = xn
    p1_ref[...] = _dot(xn, nwa_ref[...])
    p2_ref[...] = _dot(xn, nwb_ref[...]) + nb_ref[...]


_tc_node = pl.pallas_call(
    _tc_node_body,
    out_shape=(
        jax.ShapeDtypeStruct((N, H), jnp.float32),
        jax.ShapeDtypeStruct((N, HID), jnp.float32),
        jax.ShapeDtypeStruct((N, HID), jnp.float32),
    ),
)


def _tc_last_body(x_ref, a0_ref, a1_ref, w1_ref, b1_ref, w2_ref, b2_ref,
                  gam_ref, bet_ref, batch_ref, row_ref, rob_ref, out_ref):
    xn = _node_update(x_ref[...], a0_ref[...], a1_ref[...], w1_ref[...],
                      b1_ref[...], w2_ref[...], b2_ref[...], gam_ref[...],
                      bet_ref[...])
    oh = (lax.broadcasted_iota(jnp.int32, (G, 1), 0)
          == batch_ref[...]).astype(jnp.float32)
    sums = _dot(oh, xn)
    cnt = jnp.sum(oh, axis=1, keepdims=True)
    g = sums / jnp.maximum(cnt, 1.0)
    out_ref[...] = jnp.maximum(_dot(g, row_ref[...]) + rob_ref[...], 0.0)


_tc_last = pl.pallas_call(
    _tc_last_body,
    out_shape=jax.ShapeDtypeStruct((G, H), jnp.float32),
)


def _tc_prep_body(x_ref, nwa_ref, nwb_ref, nb_ref, p1_ref, p2_ref):
    p1_ref[...] = _dot(x_ref[...], nwa_ref[...])
    p2_ref[...] = _dot(x_ref[...], nwb_ref[...]) + nb_ref[...]


_tc_prep = pl.pallas_call(
    _tc_prep_body,
    out_shape=(
        jax.ShapeDtypeStruct((N, HID), jnp.float32),
        jax.ShapeDtypeStruct((N, HID), jnp.float32),
    ),
)


def kernel(x, edge_index, edge_attr, batch, e_proj_W, e_proj_b, upd_W1,
           upd_b1, upd_W2, upd_b2, conv_W1, conv_b1, conv_W2, conv_b2,
           bn_gamma, bn_beta, ro_W, ro_b):
    pad = EPAD - E
    src = jnp.concatenate([edge_index[0], jnp.zeros((pad,), jnp.int32)])
    dst = jnp.concatenate([edge_index[1], jnp.full((pad,), N, jnp.int32)])
    srcm = src.reshape(IDXROWS, 128)
    dstm = dst.reshape(IDXROWS, 128)
    ea = jnp.concatenate([edge_attr, jnp.zeros((pad, EIN), jnp.float32)])
    zeros = jnp.zeros((NP, H), jnp.float32)
    batch_row = batch.reshape(1, N)

    be = e_proj_b.reshape(1, H)
    b2 = [upd_b2[l].reshape(1, H) for l in range(DEPTH)]
    cb1 = [conv_b1[l].reshape(1, H) for l in range(DEPTH)]
    cb2 = [conv_b2[l].reshape(1, H) for l in range(DEPTH)]
    gam = [bn_gamma[l].reshape(1, H) for l in range(DEPTH)]
    bet = [bn_beta[l].reshape(1, H) for l in range(DEPTH)]
    w1a = [upd_W1[l, :H, :] for l in range(DEPTH)]
    w1b = [upd_W1[l, H:2 * H, :] for l in range(DEPTH)]
    w1c = [upd_W1[l, 2 * H:, :] for l in range(DEPTH)]
    nb1 = [upd_b1[l].reshape(1, HID) for l in range(DEPTH)]

    p1, p2 = _tc_prep(x, w1a[0], w1b[0], nb1[0])
    e = None
    out = None
    for l in range(DEPTH):
        pre = _sc_pre(p1, p2, srcm, dstm)
        if l == 0:
            e = _tc_edge0(ea, e_proj_W, be, pre, w1c[l], upd_W2[l], b2[l])
        else:
            e = _tc_edge(pre, e, w1c[l], upd_W2[l], b2[l])
        aggr = _sc_aggr(x, e, srcm, dstm, zeros)
        if l < DEPTH - 1:
            x, p1, p2 = _tc_node(x, aggr[0], aggr[1], conv_W1[l], cb1[l],
                                 conv_W2[l], cb2[l], gam[l], bet[l],
                                 w1a[l + 1], w1b[l + 1], nb1[l + 1])
        else:
            out = _tc_last(x, aggr[0], aggr[1], conv_W1[l], cb1[l],
                           conv_W2[l], cb2[l], gam[l], bet[l], batch_row,
                           ro_W, ro_b.reshape(1, H))
    return out


# speedup vs baseline: 1.6828x; 1.6828x over previous
import functools

import jax
import jax.numpy as jnp
from jax import lax
from jax.experimental import pallas as pl
from jax.experimental.pallas import tpu as pltpu
from jax.experimental.pallas import tpu_sc as plsc

N = 10000
E = 320000
H = 128
EIN = 16
DEPTH = 5
G = 64
HID = 64
BN_EPS = 1e-5

NW = 32
EPAD = 327680
EW = EPAD // NW
IDXROWS = EPAD // 128
IRW = EW // 128
CA = 512
CC = 128
NP = 10240
NPR = NP // 16

_mesh = plsc.VectorSubcoreMesh(core_axis_name="c", subcore_axis_name="s")


def _sc_pre_body(p1_hbm, p2_hbm, srcm_hbm, dstm_hbm, pre_hbm,
                 idx_s, idx_d, p1_v, p2_v, sem1, sem2):
    cid = lax.axis_index("c")
    sid = lax.axis_index("s")
    wid = sid * 2 + cid

    def chunk(k, carry):
        r0 = wid * IRW + k * (CA // 128)
        b0 = wid * EW + k * CA
        pltpu.sync_copy(srcm_hbm.at[pl.ds(r0, CA // 128)], idx_s)
        pltpu.sync_copy(dstm_hbm.at[pl.ds(r0, CA // 128)], idx_d)
        cps = []
        for j in range(CA // 128):
            cps.append(pltpu.async_copy(
                p1_hbm.at[idx_s.at[j]], p1_v.at[pl.ds(j * 128, 128)], sem1))
            cps.append(pltpu.async_copy(
                p2_hbm.at[idx_d.at[j]], p2_v.at[pl.ds(j * 128, 128)], sem2))
        for c in cps:
            c.wait()

        def addrow(r, carry2):
            for jj in range(HID // 16):
                s = pl.ds(jj * 16, 16)
                p1_v[r, s] = p1_v[r, s] + p2_v[r, s]
            return carry2
        lax.fori_loop(0, CA, addrow, 0)
        pltpu.sync_copy(p1_v, pre_hbm.at[pl.ds(b0, CA)])
        return carry
    lax.fori_loop(0, EW // CA, chunk, 0)


_sc_pre = functools.partial(
    pl.kernel,
    out_type=jax.ShapeDtypeStruct((EPAD, HID), jnp.float32),
    mesh=_mesh,
    scratch_types=[
        pltpu.VMEM((CA // 128, 128), jnp.int32),
        pltpu.VMEM((CA // 128, 128), jnp.int32),
        pltpu.VMEM((CA, HID), jnp.float32),
        pltpu.VMEM((CA, HID), jnp.float32),
        pltpu.SemaphoreType.DMA,
        pltpu.SemaphoreType.DMA,
    ],
    compiler_params=pltpu.CompilerParams(use_tc_tiling_on_sc=False),
)(_sc_pre_body)


def _sc_aggr_body(x_hbm, e_hbm, srcm_hbm, dstm_hbm, zeros_hbm, aggr_hbm,
                  idx_s, idx_d, xs_v, e_v, aggr_sh, sem1, sem2):
    cid = lax.axis_index("c")
    sid = lax.axis_index("s")
    wid = sid * 2 + cid

    pltpu.sync_copy(zeros_hbm.at[pl.ds(sid * NPR, NPR)],
                    aggr_sh.at[pl.ds(sid * NPR, NPR)])
    plsc.subcore_barrier()

    def chunk(k, carry):
        r0 = wid * IRW + k * (CC // 128)
        b0 = wid * EW + k * CC
        pltpu.sync_copy(srcm_hbm.at[pl.ds(r0, CC // 128)], idx_s)
        pltpu.sync_copy(dstm_hbm.at[pl.ds(r0, CC // 128)], idx_d)
        cps = [pltpu.async_copy(e_hbm.at[pl.ds(b0, CC)], e_v, sem2)]
        for j in range(CC // 128):
            cps.append(pltpu.async_copy(
                x_hbm.at[idx_s.at[j]], xs_v.at[pl.ds(j * 128, 128)], sem1))
        for c in cps:
            c.wait()

        def msgrow(r, carry2):
            for jj in range(H // 16):
                s = pl.ds(jj * 16, 16)
                xs_v[r, s] = jnp.maximum(xs_v[r, s] + e_v[r, s], 0.0)
            return carry2
        lax.fori_loop(0, CC, msgrow, 0)
        for j in range(CC // 128):
            pltpu.sync_copy(xs_v.at[pl.ds(j * 128, 128)],
                            aggr_sh.at[idx_d.at[j]], add=True)
        return carry
    lax.fori_loop(0, EW // CC, chunk, 0)

    plsc.subcore_barrier()
    pltpu.sync_copy(aggr_sh.at[pl.ds(sid * NPR, NPR)],
                    aggr_hbm.at[cid, pl.ds(sid * NPR, NPR)])


_sc_aggr = functools.partial(
    pl.kernel,
    out_type=jax.ShapeDtypeStruct((2, NP, H), jnp.float32),
    mesh=_mesh,
    scratch_types=[
        pltpu.VMEM((CC // 128, 128), jnp.int32),
        pltpu.VMEM((CC // 128, 128), jnp.int32),
        pltpu.VMEM((CC, H), jnp.float32),
        pltpu.VMEM((CC, H), jnp.float32),
        pltpu.VMEM_SHARED((NP, H), jnp.float32),
        pltpu.SemaphoreType.DMA,
        pltpu.SemaphoreType.DMA,
    ],
)(_sc_aggr_body)


EB = 2048


def _dot(a, b):
    return jnp.dot(a, b, preferred_element_type=jnp.float32)


def _tc_edge_body(pre_ref, e_ref, w1c_ref, w2_ref, b2_ref, eo_ref):
    e = e_ref[...]
    h = jnp.maximum(pre_ref[...] + _dot(e, w1c_ref[...]), 0.0)
    eo_ref[...] = e + _dot(h, w2_ref[...]) + b2_ref[...]


_tc_edge = pl.pallas_call(
    _tc_edge_body,
    grid=(EPAD // EB,),
    in_specs=[
        pl.BlockSpec((EB, HID), lambda i: (i, 0)),
        pl.BlockSpec((EB, H), lambda i: (i, 0)),
        pl.BlockSpec((H, HID), lambda i: (0, 0)),
        pl.BlockSpec((HID, H), lambda i: (0, 0)),
        pl.BlockSpec((1, H), lambda i: (0, 0)),
    ],
    out_specs=pl.BlockSpec((EB, H), lambda i: (i, 0)),
    out_shape=jax.ShapeDtypeStruct((EPAD, H), jnp.float32),
)


def _tc_edge0_body(ea_ref, we_ref, be_ref, pre_ref, w1c_ref, w2_ref, b2_ref,
                   eo_ref):
    e = _dot(ea_ref[...], we_ref[...]) + be_ref[...]
    h = jnp.maximum(pre_ref[...] + _dot(e, w1c_ref[...]), 0.0)
    eo_ref[...] = e + _dot(h, w2_ref[...]) + b2_ref[...]


_tc_edge0 = pl.pallas_call(
    _tc_edge0_body,
    grid=(EPAD // EB,),
    in_specs=[
        pl.BlockSpec((EB, EIN), lambda i: (i, 0)),
        pl.BlockSpec((EIN, H), lambda i: (0, 0)),
        pl.BlockSpec((1, H), lambda i: (0, 0)),
        pl.BlockSpec((EB, HID), lambda i: (i, 0)),
        pl.BlockSpec((H, HID), lambda i: (0, 0)),
        pl.BlockSpec((HID, H), lambda i: (0, 0)),
        pl.BlockSpec((1, H), lambda i: (0, 0)),
    ],
    out_specs=pl.BlockSpec((EB, H), lambda i: (i, 0)),
    out_shape=jax.ShapeDtypeStruct((EPAD, H), jnp.float32),
)


def _node_update(x, a0, a1, w1, b1, w2, b2, gam, bet):
    t = x + a0[0:N, :] + a1[0:N, :]
    u = _dot(jnp.maximum(_dot(t, w1) + b1, 0.0), w2) + b2
    mean = jnp.mean(u, axis=0, keepdims=True)
    var = jnp.mean((u - mean) * (u - mean), axis=0, keepdims=True)
    xb = (u - mean) * lax.rsqrt(var + BN_EPS) * gam + bet
    return x + jnp.maximum(xb, 0.0)


def _tc_node_body(x_ref, a0_ref, a1_ref, w1_ref, b1_ref, w2_ref, b2_ref,
                  gam_ref, bet_ref, nwa_ref, nwb_ref, nb_ref,
                  xo_ref, p1_ref, p2_ref):
    xn = _node_update(x_ref[...], a0_ref[...], a1_ref[...], w1_ref[...],
                      b1_ref[...], w2_ref[...], b2_ref[...], gam_ref[...],
                      bet_ref[...])
    xo_ref[...] = xn
    p1_ref[...] = _dot(xn, nwa_ref[...])
    p2_ref[...] = _dot(xn, nwb_ref[...]) + nb_ref[...]


_tc_node = pl.pallas_call(
    _tc_node_body,
    out_shape=(
        jax.ShapeDtypeStruct((N, H), jnp.float32),
        jax.ShapeDtypeStruct((N, HID), jnp.float32),
        jax.ShapeDtypeStruct((N, HID), jnp.float32),
    ),
)


def _tc_last_body(x_ref, a0_ref, a1_ref, w1_ref, b1_ref, w2_ref, b2_ref,
                  gam_ref, bet_ref, batch_ref, row_ref, rob_ref, out_ref):
    xn = _node_update(x_ref[...], a0_ref[...], a1_ref[...], w1_ref[...],
                      b1_ref[...], w2_ref[...], b2_ref[...], gam_ref[...],
                      bet_ref[...])
    oh = (lax.broadcasted_iota(jnp.int32, (G, 1), 0)
          == batch_ref[...]).astype(jnp.float32)
    sums = _dot(oh, xn)
    cnt = jnp.sum(oh, axis=1, keepdims=True)
    g = sums / jnp.maximum(cnt, 1.0)
    out_ref[...] = jnp.maximum(_dot(g, row_ref[...]) + rob_ref[...], 0.0)


_tc_last = pl.pallas_call(
    _tc_last_body,
    out_shape=jax.ShapeDtypeStruct((G, H), jnp.float32),
)


def _tc_prep_body(x_ref, nwa_ref, nwb_ref, nb_ref, p1_ref, p2_ref):
    p1_ref[...] = _dot(x_ref[...], nwa_ref[...])
    p2_ref[...] = _dot(x_ref[...], nwb_ref[...]) + nb_ref[...]


_tc_prep = pl.pallas_call(
    _tc_prep_body,
    out_shape=(
        jax.ShapeDtypeStruct((N, HID), jnp.float32),
        jax.ShapeDtypeStruct((N, HID), jnp.float32),
    ),
)


def kernel(x, edge_index, edge_attr, batch, e_proj_W, e_proj_b, upd_W1,
           upd_b1, upd_W2, upd_b2, conv_W1, conv_b1, conv_W2, conv_b2,
           bn_gamma, bn_beta, ro_W, ro_b):
    pad = EPAD - E
    src = jnp.concatenate([edge_index[0], jnp.zeros((pad,), jnp.int32)])
    dst = jnp.concatenate([edge_index[1], jnp.full((pad,), N, jnp.int32)])
    srcm = src.reshape(IDXROWS, 128)
    dstm = dst.reshape(IDXROWS, 128)
    ea = jnp.concatenate([edge_attr, jnp.zeros((pad, EIN), jnp.float32)])
    zeros = jnp.zeros((NP, H), jnp.float32)
    batch_row = batch.reshape(1, N)

    be = e_proj_b.reshape(1, H)
    b2 = [upd_b2[l].reshape(1, H) for l in range(DEPTH)]
    cb1 = [conv_b1[l].reshape(1, H) for l in range(DEPTH)]
    cb2 = [conv_b2[l].reshape(1, H) for l in range(DEPTH)]
    gam = [bn_gamma[l].reshape(1, H) for l in range(DEPTH)]
    bet = [bn_beta[l].reshape(1, H) for l in range(DEPTH)]
    w1a = [upd_W1[l, :H, :] for l in range(DEPTH)]
    w1b = [upd_W1[l, H:2 * H, :] for l in range(DEPTH)]
    w1c = [upd_W1[l, 2 * H:, :] for l in range(DEPTH)]
    nb1 = [upd_b1[l].reshape(1, HID) for l in range(DEPTH)]

    p1, p2 = _tc_prep(x, w1a[0], w1b[0], nb1[0])
    e = None
    out = None
    for l in range(DEPTH):
        pre = _sc_pre(p1, p2, srcm, dstm)
        if l == 0:
            e = _tc_edge0(ea, e_proj_W, be, pre, w1c[l], upd_W2[l], b2[l])
        else:
            e = _tc_edge(pre, e, w1c[l], upd_W2[l], b2[l])
        aggr = _sc_aggr(x, e, srcm, dstm, zeros)
        if l < DEPTH - 1:
            x, p1, p2 = _tc_node(x, aggr[0], aggr[1], conv_W1[l], cb1[l],
                                 conv_W2[l], cb2[l], gam[l], bet[l],
                                 w1a[l + 1], w1b[l + 1], nb1[l + 1])
        else:
            out = _tc_last(x, aggr[0], aggr[1], conv_W1[l], cb1[l],
                           conv_W2[l], cb2[l], gam[l], bet[l], batch_row,
                           ro_W, ro_b.reshape(1, H))
    return out


# speedup vs baseline: 2.3891x; 1.4198x over previous
import functools

import jax
import jax.numpy as jnp
from jax import lax
from jax.experimental import pallas as pl
from jax.experimental.pallas import tpu as pltpu
from jax.experimental.pallas import tpu_sc as plsc

N = 10000
E = 320000
H = 128
EIN = 16
DEPTH = 5
G = 64
HID = 64
BN_EPS = 1e-5

NW = 32
EPAD = 327680
EW = EPAD // NW
IDXROWS = EPAD // 128
IRW = EW // 128
CA = 256
NCH_A = EW // CA
CC = 128
NCH_C = EW // CC
NP = 10240
NPR = NP // 16

_mesh = plsc.VectorSubcoreMesh(core_axis_name="c", subcore_axis_name="s")


def _sc_gather_body(x_hbm, p2_hbm, srcm_hbm, dstm_hbm, xs_hbm, pd_hbm,
                    idx_s, idx_d, xs_v, pd_v,
                    sem_g0, sem_g1, sem_w0, sem_w1):
    cid = lax.axis_index("c")
    sid = lax.axis_index("s")
    wid = sid * 2 + cid
    IR = CA // 128
    sem_g = (sem_g0, sem_g1)
    sem_w = (sem_w0, sem_w1)
    gd = [None] * NCH_A
    wd = [None] * NCH_A
    for k in range(NCH_A + 1):
        if k < NCH_A:
            b = k % 2
            if k >= 2:
                for d in wd[k - 2]:
                    d.wait()
            r0 = wid * IRW + k * IR
            pltpu.sync_copy(srcm_hbm.at[pl.ds(r0, IR)], idx_s.at[b])
            pltpu.sync_copy(dstm_hbm.at[pl.ds(r0, IR)], idx_d.at[b])
            g = []
            for j in range(IR):
                g.append(pltpu.async_copy(
                    x_hbm.at[idx_s.at[b, j]],
                    xs_v.at[b, pl.ds(j * 128, 128)], sem_g[b]))
                g.append(pltpu.async_copy(
                    p2_hbm.at[idx_d.at[b, j]],
                    pd_v.at[b, pl.ds(j * 128, 128)], sem_g[b]))
            gd[k] = g
        if k >= 1:
            kp = k - 1
            bp = kp % 2
            for d in gd[kp]:
                d.wait()
            b0 = wid * EW + kp * CA
            wd[kp] = [
                pltpu.async_copy(xs_v.at[bp], xs_hbm.at[pl.ds(b0, CA)],
                                 sem_w[bp]),
                pltpu.async_copy(pd_v.at[bp], pd_hbm.at[pl.ds(b0, CA)],
                                 sem_w[bp]),
            ]
    for d in wd[NCH_A - 2]:
        d.wait()
    for d in wd[NCH_A - 1]:
        d.wait()


_sc_gather = functools.partial(
    pl.kernel,
    out_type=(
        jax.ShapeDtypeStruct((EPAD, H), jnp.float32),
        jax.ShapeDtypeStruct((EPAD, HID), jnp.float32),
    ),
    mesh=_mesh,
    scratch_types=[
        pltpu.VMEM((2, CA // 128, 128), jnp.int32),
        pltpu.VMEM((2, CA // 128, 128), jnp.int32),
        pltpu.VMEM((2, CA, H), jnp.float32),
        pltpu.VMEM((2, CA, HID), jnp.float32),
        pltpu.SemaphoreType.DMA,
        pltpu.SemaphoreType.DMA,
        pltpu.SemaphoreType.DMA,
        pltpu.SemaphoreType.DMA,
    ],
    compiler_params=pltpu.CompilerParams(use_tc_tiling_on_sc=False),
)(_sc_gather_body)


def _sc_aggr_body(msg_hbm, dstm_hbm, zeros_hbm, aggr_hbm,
                  idx_d, msg_v, aggr_sh, sem_m0, sem_m1, sem_i0, sem_i1):
    cid = lax.axis_index("c")
    sid = lax.axis_index("s")
    wid = sid * 2 + cid
    sem_m = (sem_m0, sem_m1)
    sem_i = (sem_i0, sem_i1)

    pltpu.sync_copy(zeros_hbm.at[pl.ds(sid * NPR, NPR)],
                    aggr_sh.at[pl.ds(sid * NPR, NPR)])
    plsc.subcore_barrier()

    md = [None] * NCH_C
    idd = [None] * NCH_C
    for k in range(NCH_C + 1):
        if k < NCH_C:
            b = k % 2
            r0 = wid * IRW + k
            b0 = wid * EW + k * CC
            idd[k] = pltpu.async_copy(dstm_hbm.at[pl.ds(r0, 1)],
                                      idx_d.at[b], sem_i[b])
            md[k] = pltpu.async_copy(msg_hbm.at[pl.ds(b0, CC)],
                                     msg_v.at[b], sem_m[b])
        if k >= 1:
            kp = k - 1
            bp = kp % 2
            idd[kp].wait()
            md[kp].wait()
            pltpu.sync_copy(msg_v.at[bp], aggr_sh.at[idx_d.at[bp, 0]],
                            add=True)

    plsc.subcore_barrier()
    pltpu.sync_copy(aggr_sh.at[pl.ds(sid * NPR, NPR)],
                    aggr_hbm.at[cid, pl.ds(sid * NPR, NPR)])


_sc_aggr = functools.partial(
    pl.kernel,
    out_type=jax.ShapeDtypeStruct((2, NP, H), jnp.float32),
    mesh=_mesh,
    scratch_types=[
        pltpu.VMEM((2, 1, 128), jnp.int32),
        pltpu.VMEM((2, CC, H), jnp.float32),
        pltpu.VMEM_SHARED((NP, H), jnp.float32),
        pltpu.SemaphoreType.DMA,
        pltpu.SemaphoreType.DMA,
        pltpu.SemaphoreType.DMA,
        pltpu.SemaphoreType.DMA,
    ],
)(_sc_aggr_body)


EB = 2048


def _dot(a, b):
    return jnp.dot(a, b, preferred_element_type=jnp.float32)


def _tc_edge_body(xs_ref, pd_ref, e_ref, w1a_ref, w1c_ref, w2_ref, b2_ref,
                  eo_ref, msg_ref):
    e = e_ref[...]
    xs = xs_ref[...]
    h = jnp.maximum(pd_ref[...] + _dot(xs, w1a_ref[...])
                    + _dot(e, w1c_ref[...]), 0.0)
    eo = e + _dot(h, w2_ref[...]) + b2_ref[...]
    eo_ref[...] = eo
    msg_ref[...] = jnp.maximum(xs + eo, 0.0)


_tc_edge = pl.pallas_call(
    _tc_edge_body,
    grid=(EPAD // EB,),
    in_specs=[
        pl.BlockSpec((EB, H), lambda i: (i, 0)),
        pl.BlockSpec((EB, HID), lambda i: (i, 0)),
        pl.BlockSpec((EB, H), lambda i: (i, 0)),
        pl.BlockSpec((H, HID), lambda i: (0, 0)),
        pl.BlockSpec((H, HID), lambda i: (0, 0)),
        pl.BlockSpec((HID, H), lambda i: (0, 0)),
        pl.BlockSpec((1, H), lambda i: (0, 0)),
    ],
    out_specs=[
        pl.BlockSpec((EB, H), lambda i: (i, 0)),
        pl.BlockSpec((EB, H), lambda i: (i, 0)),
    ],
    out_shape=(
        jax.ShapeDtypeStruct((EPAD, H), jnp.float32),
        jax.ShapeDtypeStruct((EPAD, H), jnp.float32),
    ),
)


def _tc_edge0_body(xs_ref, pd_ref, ea_ref, we_ref, be_ref, w1a_ref, w1c_ref,
                   w2_ref, b2_ref, eo_ref, msg_ref):
    e = _dot(ea_ref[...], we_ref[...]) + be_ref[...]
    xs = xs_ref[...]
    h = jnp.maximum(pd_ref[...] + _dot(xs, w1a_ref[...])
                    + _dot(e, w1c_ref[...]), 0.0)
    eo = e + _dot(h, w2_ref[...]) + b2_ref[...]
    eo_ref[...] = eo
    msg_ref[...] = jnp.maximum(xs + eo, 0.0)


_tc_edge0 = pl.pallas_call(
    _tc_edge0_body,
    grid=(EPAD // EB,),
    in_specs=[
        pl.BlockSpec((EB, H), lambda i: (i, 0)),
        pl.BlockSpec((EB, HID), lambda i: (i, 0)),
        pl.BlockSpec((EB, EIN), lambda i: (i, 0)),
        pl.BlockSpec((EIN, H), lambda i: (0, 0)),
        pl.BlockSpec((1, H), lambda i: (0, 0)),
        pl.BlockSpec((H, HID), lambda i: (0, 0)),
        pl.BlockSpec((H, HID), lambda i: (0, 0)),
        pl.BlockSpec((HID, H), lambda i: (0, 0)),
        pl.BlockSpec((1, H), lambda i: (0, 0)),
    ],
    out_specs=[
        pl.BlockSpec((EB, H), lambda i: (i, 0)),
        pl.BlockSpec((EB, H), lambda i: (i, 0)),
    ],
    out_shape=(
        jax.ShapeDtypeStruct((EPAD, H), jnp.float32),
        jax.ShapeDtypeStruct((EPAD, H), jnp.float32),
    ),
)


def _node_update(x, a0, a1, w1, b1, w2, b2, gam, bet):
    t = x + a0[0:N, :] + a1[0:N, :]
    u = _dot(jnp.maximum(_dot(t, w1) + b1, 0.0), w2) + b2
    mean = jnp.mean(u, axis=0, keepdims=True)
    var = jnp.mean((u - mean) * (u - mean), axis=0, keepdims=True)
    xb = (u - mean) * lax.rsqrt(var + BN_EPS) * gam + bet
    return x + jnp.maximum(xb, 0.0)


def _tc_node_body(x_ref, a0_ref, a1_ref, w1_ref, b1_ref, w2_ref, b2_ref,
                  gam_ref, bet_ref, nwb_ref, nb_ref, xo_ref, p2_ref):
    xn = _node_update(x_ref[...], a0_ref[...], a1_ref[...], w1_ref[...],
                      b1_ref[...], w2_ref[...], b2_ref[...], gam_ref[...],
                      bet_ref[...])
    xo_ref[...] = xn
    p2_ref[...] = _dot(xn, nwb_ref[...]) + nb_ref[...]


_tc_node = pl.pallas_call(
    _tc_node_body,
    out_shape=(
        jax.ShapeDtypeStruct((N, H), jnp.float32),
        jax.ShapeDtypeStruct((N, HID), jnp.float32),
    ),
)


def _tc_last_body(x_ref, a0_ref, a1_ref, w1_ref, b1_ref, w2_ref, b2_ref,
                  gam_ref, bet_ref, batch_ref, row_ref, rob_ref, out_ref):
    xn = _node_update(x_ref[...], a0_ref[...], a1_ref[...], w1_ref[...],
                      b1_ref[...], w2_ref[...], b2_ref[...], gam_ref[...],
                      bet_ref[...])
    oh = (lax.broadcasted_iota(jnp.int32, (G, 1), 0)
          == batch_ref[...]).astype(jnp.float32)
    sums = _dot(oh, xn)
    cnt = jnp.sum(oh, axis=1, keepdims=True)
    g = sums / jnp.maximum(cnt, 1.0)
    out_ref[...] = jnp.maximum(_dot(g, row_ref[...]) + rob_ref[...], 0.0)


_tc_last = pl.pallas_call(
    _tc_last_body,
    out_shape=jax.ShapeDtypeStruct((G, H), jnp.float32),
)


def _tc_prep_body(x_ref, nwb_ref, nb_ref, p2_ref):
    p2_ref[...] = _dot(x_ref[...], nwb_ref[...]) + nb_ref[...]


_tc_prep = pl.pallas_call(
    _tc_prep_body,
    out_shape=jax.ShapeDtypeStruct((N, HID), jnp.float32),
)


def kernel(x, edge_index, edge_attr, batch, e_proj_W, e_proj_b, upd_W1,
           upd_b1, upd_W2, upd_b2, conv_W1, conv_b1, conv_W2, conv_b2,
           bn_gamma, bn_beta, ro_W, ro_b):
    pad = EPAD - E
    src = jnp.concatenate([edge_index[0], jnp.zeros((pad,), jnp.int32)])
    dst = jnp.concatenate([edge_index[1], jnp.full((pad,), N, jnp.int32)])
    srcm = src.reshape(IDXROWS, 128)
    dstm = dst.reshape(IDXROWS, 128)
    ea = jnp.concatenate([edge_attr, jnp.zeros((pad, EIN), jnp.float32)])
    zeros = jnp.zeros((NP, H), jnp.float32)
    batch_row = batch.reshape(1, N)

    be = e_proj_b.reshape(1, H)
    b2 = [upd_b2[l].reshape(1, H) for l in range(DEPTH)]
    cb1 = [conv_b1[l].reshape(1, H) for l in range(DEPTH)]
    cb2 = [conv_b2[l].reshape(1, H) for l in range(DEPTH)]
    gam = [bn_gamma[l].reshape(1, H) for l in range(DEPTH)]
    bet = [bn_beta[l].reshape(1, H) for l in range(DEPTH)]
    w1a = [upd_W1[l, :H, :] for l in range(DEPTH)]
    w1b = [upd_W1[l, H:2 * H, :] for l in range(DEPTH)]
    w1c = [upd_W1[l, 2 * H:, :] for l in range(DEPTH)]
    nb1 = [upd_b1[l].reshape(1, HID) for l in range(DEPTH)]

    p2 = _tc_prep(x, w1b[0], nb1[0])
    e = None
    out = None
    for l in range(DEPTH):
        xs, pd = _sc_gather(x, p2, srcm, dstm)
        if l == 0:
            e, msg = _tc_edge0(xs, pd, ea, e_proj_W, be, w1a[l], w1c[l],
                               upd_W2[l], b2[l])
        else:
            e, msg = _tc_edge(xs, pd, e, w1a[l], w1c[l], upd_W2[l], b2[l])
        aggr = _sc_aggr(msg, dstm, zeros)
        if l < DEPTH - 1:
            x, p2 = _tc_node(x, aggr[0], aggr[1], conv_W1[l], cb1[l],
                             conv_W2[l], cb2[l], gam[l], bet[l],
                             w1b[l + 1], nb1[l + 1])
        else:
            out = _tc_last(x, aggr[0], aggr[1], conv_W1[l], cb1[l],
                           conv_W2[l], cb2[l], gam[l], bet[l], batch_row,
                           ro_W, ro_b.reshape(1, H))
    return out


# speedup vs baseline: 2.3945x; 1.0023x over previous
import functools

import jax
import jax.numpy as jnp
from jax import lax
from jax.experimental import pallas as pl
from jax.experimental.pallas import tpu as pltpu
from jax.experimental.pallas import tpu_sc as plsc

N = 10000
E = 320000
H = 128
EIN = 16
DEPTH = 5
G = 64
HID = 64
BN_EPS = 1e-5

NW = 32
EPAD = 327680
EW = EPAD // NW
IDXROWS = EPAD // 128
IRW = EW // 128
CA = 128
NCH_A = EW // CA
DA = 4
CC = 128
NCH_C = EW // CC
DC = 2
NP = 10240
NPR = NP // 16

_mesh = plsc.VectorSubcoreMesh(core_axis_name="c", subcore_axis_name="s")


def _sc_gather_body(x_hbm, p2_hbm, srcm_hbm, dstm_hbm, xs_hbm, pd_hbm,
                    idx_s, idx_d, xs_v, pd_v,
                    sem_i, sem_g0, sem_g1, sem_g2, sem_g3,
                    sem_w0, sem_w1, sem_w2, sem_w3):
    cid = lax.axis_index("c")
    sid = lax.axis_index("s")
    wid = sid * 2 + cid
    sem_g = (sem_g0, sem_g1, sem_g2, sem_g3)
    sem_w = (sem_w0, sem_w1, sem_w2, sem_w3)

    r0 = wid * IRW
    i0 = pltpu.async_copy(srcm_hbm.at[pl.ds(r0, IRW)], idx_s, sem_i)
    i1 = pltpu.async_copy(dstm_hbm.at[pl.ds(r0, IRW)], idx_d, sem_i)
    i0.wait()
    i1.wait()

    LAG = DA - 1
    gd = [None] * NCH_A
    wd = [None] * NCH_A
    for k in range(NCH_A + LAG):
        if k < NCH_A:
            b = k % DA
            if k >= DA:
                for d in wd[k - DA]:
                    d.wait()
            gd[k] = [
                pltpu.async_copy(x_hbm.at[idx_s.at[k]],
                                 xs_v.at[b], sem_g[b]),
                pltpu.async_copy(p2_hbm.at[idx_d.at[k]],
                                 pd_v.at[b], sem_g[b]),
            ]
        if k >= LAG:
            kp = k - LAG
            bp = kp % DA
            for d in gd[kp]:
                d.wait()
            b0 = wid * EW + kp * CA
            wd[kp] = [
                pltpu.async_copy(xs_v.at[bp], xs_hbm.at[pl.ds(b0, CA)],
                                 sem_w[bp]),
                pltpu.async_copy(pd_v.at[bp], pd_hbm.at[pl.ds(b0, CA)],
                                 sem_w[bp]),
            ]
    for k in range(NCH_A - DA, NCH_A):
        for d in wd[k]:
            d.wait()


_sc_gather = functools.partial(
    pl.kernel,
    out_type=(
        jax.ShapeDtypeStruct((EPAD, H), jnp.float32),
        jax.ShapeDtypeStruct((EPAD, HID), jnp.float32),
    ),
    mesh=_mesh,
    scratch_types=[
        pltpu.VMEM((IRW, 128), jnp.int32),
        pltpu.VMEM((IRW, 128), jnp.int32),
        pltpu.VMEM((DA, CA, H), jnp.float32),
        pltpu.VMEM((DA, CA, HID), jnp.float32),
        pltpu.SemaphoreType.DMA,
        pltpu.SemaphoreType.DMA,
        pltpu.SemaphoreType.DMA,
        pltpu.SemaphoreType.DMA,
        pltpu.SemaphoreType.DMA,
        pltpu.SemaphoreType.DMA,
        pltpu.SemaphoreType.DMA,
        pltpu.SemaphoreType.DMA,
        pltpu.SemaphoreType.DMA,
    ],
    compiler_params=pltpu.CompilerParams(use_tc_tiling_on_sc=False),
)(_sc_gather_body)


def _sc_aggr_body(msg_hbm, dstm_hbm, zeros_hbm, aggr_hbm,
                  idx_d, msg_v, aggr_sh, sem_m0, sem_m1, sem_i0, sem_i1):
    cid = lax.axis_index("c")
    sid = lax.axis_index("s")
    wid = sid * 2 + cid
    sem_m = (sem_m0, sem_m1)
    sem_i = (sem_i0, sem_i1)

    zd = pltpu.async_copy(zeros_hbm.at[pl.ds(sid * NPR, NPR)],
                          aggr_sh.at[pl.ds(sid * NPR, NPR)], sem_i[0])
    pltpu.sync_copy(dstm_hbm.at[pl.ds(wid * IRW, IRW)], idx_d)
    zd.wait()
    plsc.subcore_barrier()

    md = [None] * NCH_C
    for k in range(NCH_C + 1):
        if k < NCH_C:
            b = k % 2
            b0 = wid * EW + k * CC
            md[k] = pltpu.async_copy(msg_hbm.at[pl.ds(b0, CC)],
                                     msg_v.at[b], sem_m[b])
        if k >= 1:
            kp = k - 1
            bp = kp % 2
            md[kp].wait()
            pltpu.sync_copy(msg_v.at[bp], aggr_sh.at[idx_d.at[kp]],
                            add=True)

    plsc.subcore_barrier()
    pltpu.sync_copy(aggr_sh.at[pl.ds(sid * NPR, NPR)],
                    aggr_hbm.at[cid, pl.ds(sid * NPR, NPR)])


_sc_aggr = functools.partial(
    pl.kernel,
    out_type=jax.ShapeDtypeStruct((2, NP, H), jnp.float32),
    mesh=_mesh,
    scratch_types=[
        pltpu.VMEM((IRW, 128), jnp.int32),
        pltpu.VMEM((2, CC, H), jnp.float32),
        pltpu.VMEM_SHARED((NP, H), jnp.float32),
        pltpu.SemaphoreType.DMA,
        pltpu.SemaphoreType.DMA,
        pltpu.SemaphoreType.DMA,
        pltpu.SemaphoreType.DMA,
    ],
)(_sc_aggr_body)


EB = 2048


def _dot(a, b):
    return jnp.dot(a, b, preferred_element_type=jnp.float32)


def _tc_edge_body(xs_ref, pd_ref, e_ref, w1a_ref, w1c_ref, w2_ref, b2_ref,
                  eo_ref, msg_ref):
    e = e_ref[...]
    xs = xs_ref[...]
    h = jnp.maximum(pd_ref[...] + _dot(xs, w1a_ref[...])
                    + _dot(e, w1c_ref[...]), 0.0)
    eo = e + _dot(h, w2_ref[...]) + b2_ref[...]
    eo_ref[...] = eo
    msg_ref[...] = jnp.maximum(xs + eo, 0.0)


_tc_edge = pl.pallas_call(
    _tc_edge_body,
    grid=(EPAD // EB,),
    in_specs=[
        pl.BlockSpec((EB, H), lambda i: (i, 0)),
        pl.BlockSpec((EB, HID), lambda i: (i, 0)),
        pl.BlockSpec((EB, H), lambda i: (i, 0)),
        pl.BlockSpec((H, HID), lambda i: (0, 0)),
        pl.BlockSpec((H, HID), lambda i: (0, 0)),
        pl.BlockSpec((HID, H), lambda i: (0, 0)),
        pl.BlockSpec((1, H), lambda i: (0, 0)),
    ],
    out_specs=[
        pl.BlockSpec((EB, H), lambda i: (i, 0)),
        pl.BlockSpec((EB, H), lambda i: (i, 0)),
    ],
    out_shape=(
        jax.ShapeDtypeStruct((EPAD, H), jnp.float32),
        jax.ShapeDtypeStruct((EPAD, H), jnp.float32),
    ),
)


def _tc_edge0_body(xs_ref, pd_ref, ea_ref, we_ref, be_ref, w1a_ref, w1c_ref,
                   w2_ref, b2_ref, eo_ref, msg_ref):
    e = _dot(ea_ref[...], we_ref[...]) + be_ref[...]
    xs = xs_ref[...]
    h = jnp.maximum(pd_ref[...] + _dot(xs, w1a_ref[...])
                    + _dot(e, w1c_ref[...]), 0.0)
    eo = e + _dot(h, w2_ref[...]) + b2_ref[...]
    eo_ref[...] = eo
    msg_ref[...] = jnp.maximum(xs + eo, 0.0)


_tc_edge0 = pl.pallas_call(
    _tc_edge0_body,
    grid=(EPAD // EB,),
    in_specs=[
        pl.BlockSpec((EB, H), lambda i: (i, 0)),
        pl.BlockSpec((EB, HID), lambda i: (i, 0)),
        pl.BlockSpec((EB, EIN), lambda i: (i, 0)),
        pl.BlockSpec((EIN, H), lambda i: (0, 0)),
        pl.BlockSpec((1, H), lambda i: (0, 0)),
        pl.BlockSpec((H, HID), lambda i: (0, 0)),
        pl.BlockSpec((H, HID), lambda i: (0, 0)),
        pl.BlockSpec((HID, H), lambda i: (0, 0)),
        pl.BlockSpec((1, H), lambda i: (0, 0)),
    ],
    out_specs=[
        pl.BlockSpec((EB, H), lambda i: (i, 0)),
        pl.BlockSpec((EB, H), lambda i: (i, 0)),
    ],
    out_shape=(
        jax.ShapeDtypeStruct((EPAD, H), jnp.float32),
        jax.ShapeDtypeStruct((EPAD, H), jnp.float32),
    ),
)


def _node_update(x, a0, a1, w1, b1, w2, b2, gam, bet):
    t = x + a0[0:N, :] + a1[0:N, :]
    u = _dot(jnp.maximum(_dot(t, w1) + b1, 0.0), w2) + b2
    mean = jnp.mean(u, axis=0, keepdims=True)
    var = jnp.mean((u - mean) * (u - mean), axis=0, keepdims=True)
    xb = (u - mean) * lax.rsqrt(var + BN_EPS) * gam + bet
    return x + jnp.maximum(xb, 0.0)


def _tc_node_body(x_ref, a0_ref, a1_ref, w1_ref, b1_ref, w2_ref, b2_ref,
                  gam_ref, bet_ref, nwb_ref, nb_ref, xo_ref, p2_ref):
    xn = _node_update(x_ref[...], a0_ref[...], a1_ref[...], w1_ref[...],
                      b1_ref[...], w2_ref[...], b2_ref[...], gam_ref[...],
                      bet_ref[...])
    xo_ref[...] = xn
    p2_ref[...] = _dot(xn, nwb_ref[...]) + nb_ref[...]


_tc_node = pl.pallas_call(
    _tc_node_body,
    out_shape=(
        jax.ShapeDtypeStruct((N, H), jnp.float32),
        jax.ShapeDtypeStruct((N, HID), jnp.float32),
    ),
)


def _tc_last_body(x_ref, a0_ref, a1_ref, w1_ref, b1_ref, w2_ref, b2_ref,
                  gam_ref, bet_ref, batch_ref, row_ref, rob_ref, out_ref):
    xn = _node_update(x_ref[...], a0_ref[...], a1_ref[...], w1_ref[...],
                      b1_ref[...], w2_ref[...], b2_ref[...], gam_ref[...],
                      bet_ref[...])
    oh = (lax.broadcasted_iota(jnp.int32, (G, 1), 0)
          == batch_ref[...]).astype(jnp.float32)
    sums = _dot(oh, xn)
    cnt = jnp.sum(oh, axis=1, keepdims=True)
    g = sums / jnp.maximum(cnt, 1.0)
    out_ref[...] = jnp.maximum(_dot(g, row_ref[...]) + rob_ref[...], 0.0)


_tc_last = pl.pallas_call(
    _tc_last_body,
    out_shape=jax.ShapeDtypeStruct((G, H), jnp.float32),
)


def _tc_prep_body(x_ref, nwb_ref, nb_ref, p2_ref):
    p2_ref[...] = _dot(x_ref[...], nwb_ref[...]) + nb_ref[...]


_tc_prep = pl.pallas_call(
    _tc_prep_body,
    out_shape=jax.ShapeDtypeStruct((N, HID), jnp.float32),
)


def kernel(x, edge_index, edge_attr, batch, e_proj_W, e_proj_b, upd_W1,
           upd_b1, upd_W2, upd_b2, conv_W1, conv_b1, conv_W2, conv_b2,
           bn_gamma, bn_beta, ro_W, ro_b):
    pad = EPAD - E
    src = jnp.concatenate([edge_index[0], jnp.zeros((pad,), jnp.int32)])
    dst = jnp.concatenate([edge_index[1], jnp.full((pad,), N, jnp.int32)])
    srcm = src.reshape(IDXROWS, 128)
    dstm = dst.reshape(IDXROWS, 128)
    ea = jnp.concatenate([edge_attr, jnp.zeros((pad, EIN), jnp.float32)])
    zeros = jnp.zeros((NP, H), jnp.float32)
    batch_row = batch.reshape(1, N)

    be = e_proj_b.reshape(1, H)
    b2 = [upd_b2[l].reshape(1, H) for l in range(DEPTH)]
    cb1 = [conv_b1[l].reshape(1, H) for l in range(DEPTH)]
    cb2 = [conv_b2[l].reshape(1, H) for l in range(DEPTH)]
    gam = [bn_gamma[l].reshape(1, H) for l in range(DEPTH)]
    bet = [bn_beta[l].reshape(1, H) for l in range(DEPTH)]
    w1a = [upd_W1[l, :H, :] for l in range(DEPTH)]
    w1b = [upd_W1[l, H:2 * H, :] for l in range(DEPTH)]
    w1c = [upd_W1[l, 2 * H:, :] for l in range(DEPTH)]
    nb1 = [upd_b1[l].reshape(1, HID) for l in range(DEPTH)]

    p2 = _tc_prep(x, w1b[0], nb1[0])
    e = None
    out = None
    for l in range(DEPTH):
        xs, pd = _sc_gather(x, p2, srcm, dstm)
        if l == 0:
            e, msg = _tc_edge0(xs, pd, ea, e_proj_W, be, w1a[l], w1c[l],
                               upd_W2[l], b2[l])
        else:
            e, msg = _tc_edge(xs, pd, e, w1a[l], w1c[l], upd_W2[l], b2[l])
        aggr = _sc_aggr(msg, dstm, zeros)
        if l < DEPTH - 1:
            x, p2 = _tc_node(x, aggr[0], aggr[1], conv_W1[l], cb1[l],
                             conv_W2[l], cb2[l], gam[l], bet[l],
                             w1b[l + 1], nb1[l + 1])
        else:
            out = _tc_last(x, aggr[0], aggr[1], conv_W1[l], cb1[l],
                           conv_W2[l], cb2[l], gam[l], bet[l], batch_row,
                           ro_W, ro_b.reshape(1, H))
    return out


# speedup vs baseline: 2.5724x; 1.0743x over previous
import functools

import jax
import jax.numpy as jnp
from jax import lax
from jax.experimental import pallas as pl
from jax.experimental.pallas import tpu as pltpu
from jax.experimental.pallas import tpu_sc as plsc

N = 10000
E = 320000
H = 128
EIN = 16
DEPTH = 5
G = 64
HID = 64
BN_EPS = 1e-5

NW = 32
EPAD = 327680
EW = EPAD // NW
IDXROWS = EPAD // 128
IRW = EW // 128
CA = 128
NCH_A = EW // CA
DA = 4
CC = 128
NCH_C = EW // CC
DC = 2
NP = 10240
NPR = NP // 16

_mesh = plsc.VectorSubcoreMesh(core_axis_name="c", subcore_axis_name="s")


def _sc_gather_body(x_hbm, p2_hbm, srcm_hbm, dstm_hbm, xs_hbm, pd_hbm,
                    idx_s, idx_d, xs_v, pd_v,
                    sem_i, sem_g0, sem_g1, sem_g2, sem_g3,
                    sem_w0, sem_w1, sem_w2, sem_w3):
    cid = lax.axis_index("c")
    sid = lax.axis_index("s")
    wid = sid * 2 + cid
    sem_g = (sem_g0, sem_g1, sem_g2, sem_g3)
    sem_w = (sem_w0, sem_w1, sem_w2, sem_w3)

    r0 = wid * IRW
    i0 = pltpu.async_copy(srcm_hbm.at[pl.ds(r0, IRW)], idx_s, sem_i)
    i1 = pltpu.async_copy(dstm_hbm.at[pl.ds(r0, IRW)], idx_d, sem_i)
    i0.wait()
    i1.wait()

    LAG = DA - 1
    gd = [None] * NCH_A
    wd = [None] * NCH_A
    for k in range(NCH_A + LAG):
        if k < NCH_A:
            b = k % DA
            if k >= DA:
                for d in wd[k - DA]:
                    d.wait()
            gd[k] = [
                pltpu.async_copy(x_hbm.at[idx_s.at[k]],
                                 xs_v.at[b], sem_g[b]),
                pltpu.async_copy(p2_hbm.at[idx_d.at[k]],
                                 pd_v.at[b], sem_g[b]),
            ]
        if k >= LAG:
            kp = k - LAG
            bp = kp % DA
            for d in gd[kp]:
                d.wait()
            b0 = wid * EW + kp * CA
            wd[kp] = [
                pltpu.async_copy(xs_v.at[bp], xs_hbm.at[pl.ds(b0, CA)],
                                 sem_w[bp]),
                pltpu.async_copy(pd_v.at[bp], pd_hbm.at[pl.ds(b0, CA)],
                                 sem_w[bp]),
            ]
    for k in range(NCH_A - DA, NCH_A):
        for d in wd[k]:
            d.wait()


_sc_gather = functools.partial(
    pl.kernel,
    out_type=(
        jax.ShapeDtypeStruct((EPAD, H // 2), jnp.int32),
        jax.ShapeDtypeStruct((EPAD, HID // 2), jnp.int32),
    ),
    mesh=_mesh,
    scratch_types=[
        pltpu.VMEM((IRW, 128), jnp.int32),
        pltpu.VMEM((IRW, 128), jnp.int32),
        pltpu.VMEM((DA, CA, H // 2), jnp.int32),
        pltpu.VMEM((DA, CA, HID // 2), jnp.int32),
        pltpu.SemaphoreType.DMA,
        pltpu.SemaphoreType.DMA,
        pltpu.SemaphoreType.DMA,
        pltpu.SemaphoreType.DMA,
        pltpu.SemaphoreType.DMA,
        pltpu.SemaphoreType.DMA,
        pltpu.SemaphoreType.DMA,
        pltpu.SemaphoreType.DMA,
        pltpu.SemaphoreType.DMA,
    ],
    compiler_params=pltpu.CompilerParams(use_tc_tiling_on_sc=False),
)(_sc_gather_body)


def _sc_aggr_body(msg_hbm, dstm_hbm, zeros_hbm, aggr_hbm,
                  idx_d, msg_v, aggr_sh, sem_m0, sem_m1, sem_i0, sem_i1):
    cid = lax.axis_index("c")
    sid = lax.axis_index("s")
    wid = sid * 2 + cid
    sem_m = (sem_m0, sem_m1)
    sem_i = (sem_i0, sem_i1)

    zd = pltpu.async_copy(zeros_hbm.at[pl.ds(sid * NPR, NPR)],
                          aggr_sh.at[pl.ds(sid * NPR, NPR)], sem_i[0])
    pltpu.sync_copy(dstm_hbm.at[pl.ds(wid * IRW, IRW)], idx_d)
    zd.wait()
    plsc.subcore_barrier()

    md = [None] * NCH_C
    for k in range(NCH_C + 1):
        if k < NCH_C:
            b = k % 2
            b0 = wid * EW + k * CC
            md[k] = pltpu.async_copy(msg_hbm.at[pl.ds(b0, CC)],
                                     msg_v.at[b], sem_m[b])
        if k >= 1:
            kp = k - 1
            bp = kp % 2
            md[kp].wait()
            pltpu.sync_copy(msg_v.at[bp], aggr_sh.at[idx_d.at[kp]],
                            add=True)

    plsc.subcore_barrier()
    pltpu.sync_copy(aggr_sh.at[pl.ds(sid * NPR, NPR)],
                    aggr_hbm.at[cid, pl.ds(sid * NPR, NPR)])


_sc_aggr = functools.partial(
    pl.kernel,
    out_type=jax.ShapeDtypeStruct((2, NP, H), jnp.float32),
    mesh=_mesh,
    scratch_types=[
        pltpu.VMEM((IRW, 128), jnp.int32),
        pltpu.VMEM((2, CC, H), jnp.float32),
        pltpu.VMEM_SHARED((NP, H), jnp.float32),
        pltpu.SemaphoreType.DMA,
        pltpu.SemaphoreType.DMA,
        pltpu.SemaphoreType.DMA,
        pltpu.SemaphoreType.DMA,
    ],
)(_sc_aggr_body)


EB = 2048


def _dot(a, b):
    return jnp.dot(a, b, preferred_element_type=jnp.float32)


def _rtne(v):
    b = lax.bitcast_convert_type(v, jnp.int32)
    r = b + jnp.int32(0x7FFF) + lax.shift_right_logical(b, 16).astype(jnp.int32) % 2
    return r & (-65536)


def _pack(v, f):
    ra = _rtne(v[:, :f])
    rb = _rtne(v[:, f:])
    return ra | lax.shift_right_logical(rb, 16).astype(jnp.int32)


def _unpack(p):
    hi = lax.bitcast_convert_type(p & (-65536), jnp.float32)
    lo = lax.bitcast_convert_type(lax.shift_left(p, 16), jnp.float32)
    return jnp.concatenate([hi, lo], axis=1)


def _tc_edge_body(xs_ref, pd_ref, e_ref, w1a_ref, w1c_ref, w2_ref, b2_ref,
                  eo_ref, msg_ref):
    e = _unpack(e_ref[...])
    xs = _unpack(xs_ref[...])
    h = jnp.maximum(_unpack(pd_ref[...]) + _dot(xs, w1a_ref[...])
                    + _dot(e, w1c_ref[...]), 0.0)
    eo = e + _dot(h, w2_ref[...]) + b2_ref[...]
    eo_ref[...] = _pack(eo, H // 2)
    msg_ref[...] = jnp.maximum(xs + eo, 0.0)


_tc_edge = pl.pallas_call(
    _tc_edge_body,
    grid=(EPAD // EB,),
    in_specs=[
        pl.BlockSpec((EB, H // 2), lambda i: (i, 0)),
        pl.BlockSpec((EB, HID // 2), lambda i: (i, 0)),
        pl.BlockSpec((EB, H // 2), lambda i: (i, 0)),
        pl.BlockSpec((H, HID), lambda i: (0, 0)),
        pl.BlockSpec((H, HID), lambda i: (0, 0)),
        pl.BlockSpec((HID, H), lambda i: (0, 0)),
        pl.BlockSpec((1, H), lambda i: (0, 0)),
    ],
    out_specs=[
        pl.BlockSpec((EB, H // 2), lambda i: (i, 0)),
        pl.BlockSpec((EB, H), lambda i: (i, 0)),
    ],
    out_shape=(
        jax.ShapeDtypeStruct((EPAD, H // 2), jnp.int32),
        jax.ShapeDtypeStruct((EPAD, H), jnp.float32),
    ),
)


def _tc_edge0_body(xs_ref, pd_ref, ea_ref, we_ref, be_ref, w1a_ref, w1c_ref,
                   w2_ref, b2_ref, eo_ref, msg_ref):
    e = _dot(ea_ref[...], we_ref[...]) + be_ref[...]
    xs = _unpack(xs_ref[...])
    h = jnp.maximum(_unpack(pd_ref[...]) + _dot(xs, w1a_ref[...])
                    + _dot(e, w1c_ref[...]), 0.0)
    eo = e + _dot(h, w2_ref[...]) + b2_ref[...]
    eo_ref[...] = _pack(eo, H // 2)
    msg_ref[...] = jnp.maximum(xs + eo, 0.0)


_tc_edge0 = pl.pallas_call(
    _tc_edge0_body,
    grid=(EPAD // EB,),
    in_specs=[
        pl.BlockSpec((EB, H // 2), lambda i: (i, 0)),
        pl.BlockSpec((EB, HID // 2), lambda i: (i, 0)),
        pl.BlockSpec((EB, EIN), lambda i: (i, 0)),
        pl.BlockSpec((EIN, H), lambda i: (0, 0)),
        pl.BlockSpec((1, H), lambda i: (0, 0)),
        pl.BlockSpec((H, HID), lambda i: (0, 0)),
        pl.BlockSpec((H, HID), lambda i: (0, 0)),
        pl.BlockSpec((HID, H), lambda i: (0, 0)),
        pl.BlockSpec((1, H), lambda i: (0, 0)),
    ],
    out_specs=[
        pl.BlockSpec((EB, H // 2), lambda i: (i, 0)),
        pl.BlockSpec((EB, H), lambda i: (i, 0)),
    ],
    out_shape=(
        jax.ShapeDtypeStruct((EPAD, H // 2), jnp.int32),
        jax.ShapeDtypeStruct((EPAD, H), jnp.float32),
    ),
)


def _node_update(x, a0, a1, w1, b1, w2, b2, gam, bet):
    t = x + a0[0:N, :] + a1[0:N, :]
    u = _dot(jnp.maximum(_dot(t, w1) + b1, 0.0), w2) + b2
    mean = jnp.mean(u, axis=0, keepdims=True)
    var = jnp.mean((u - mean) * (u - mean), axis=0, keepdims=True)
    xb = (u - mean) * lax.rsqrt(var + BN_EPS) * gam + bet
    return x + jnp.maximum(xb, 0.0)


def _tc_node_body(x_ref, a0_ref, a1_ref, w1_ref, b1_ref, w2_ref, b2_ref,
                  gam_ref, bet_ref, nwb_ref, nb_ref, xo_ref, xp_ref, p2_ref):
    xn = _node_update(x_ref[...], a0_ref[...], a1_ref[...], w1_ref[...],
                      b1_ref[...], w2_ref[...], b2_ref[...], gam_ref[...],
                      bet_ref[...])
    xo_ref[...] = xn
    xp_ref[...] = _pack(xn, H // 2)
    p2_ref[...] = _pack(_dot(xn, nwb_ref[...]) + nb_ref[...], HID // 2)


_tc_node = pl.pallas_call(
    _tc_node_body,
    out_shape=(
        jax.ShapeDtypeStruct((N, H), jnp.float32),
        jax.ShapeDtypeStruct((N, H // 2), jnp.int32),
        jax.ShapeDtypeStruct((N, HID // 2), jnp.int32),
    ),
)


def _tc_last_body(x_ref, a0_ref, a1_ref, w1_ref, b1_ref, w2_ref, b2_ref,
                  gam_ref, bet_ref, batch_ref, row_ref, rob_ref, out_ref):
    xn = _node_update(x_ref[...], a0_ref[...], a1_ref[...], w1_ref[...],
                      b1_ref[...], w2_ref[...], b2_ref[...], gam_ref[...],
                      bet_ref[...])
    oh = (lax.broadcasted_iota(jnp.int32, (G, 1), 0)
          == batch_ref[...]).astype(jnp.float32)
    sums = _dot(oh, xn)
    cnt = jnp.sum(oh, axis=1, keepdims=True)
    g = sums / jnp.maximum(cnt, 1.0)
    out_ref[...] = jnp.maximum(_dot(g, row_ref[...]) + rob_ref[...], 0.0)


_tc_last = pl.pallas_call(
    _tc_last_body,
    out_shape=jax.ShapeDtypeStruct((G, H), jnp.float32),
)


def _tc_prep_body(x_ref, nwb_ref, nb_ref, xp_ref, p2_ref):
    xp_ref[...] = _pack(x_ref[...], H // 2)
    p2_ref[...] = _pack(_dot(x_ref[...], nwb_ref[...]) + nb_ref[...], HID // 2)


_tc_prep = pl.pallas_call(
    _tc_prep_body,
    out_shape=(
        jax.ShapeDtypeStruct((N, H // 2), jnp.int32),
        jax.ShapeDtypeStruct((N, HID // 2), jnp.int32),
    ),
)


def kernel(x, edge_index, edge_attr, batch, e_proj_W, e_proj_b, upd_W1,
           upd_b1, upd_W2, upd_b2, conv_W1, conv_b1, conv_W2, conv_b2,
           bn_gamma, bn_beta, ro_W, ro_b):
    pad = EPAD - E
    src = jnp.concatenate([edge_index[0], jnp.zeros((pad,), jnp.int32)])
    dst = jnp.concatenate([edge_index[1], jnp.full((pad,), N, jnp.int32)])
    srcm = src.reshape(IDXROWS, 128)
    dstm = dst.reshape(IDXROWS, 128)
    ea = jnp.concatenate([edge_attr, jnp.zeros((pad, EIN), jnp.float32)])
    zeros = jnp.zeros((NP, H), jnp.float32)
    batch_row = batch.reshape(1, N)

    be = e_proj_b.reshape(1, H)
    b2 = [upd_b2[l].reshape(1, H) for l in range(DEPTH)]
    cb1 = [conv_b1[l].reshape(1, H) for l in range(DEPTH)]
    cb2 = [conv_b2[l].reshape(1, H) for l in range(DEPTH)]
    gam = [bn_gamma[l].reshape(1, H) for l in range(DEPTH)]
    bet = [bn_beta[l].reshape(1, H) for l in range(DEPTH)]
    w1a = [upd_W1[l, :H, :] for l in range(DEPTH)]
    w1b = [upd_W1[l, H:2 * H, :] for l in range(DEPTH)]
    w1c = [upd_W1[l, 2 * H:, :] for l in range(DEPTH)]
    nb1 = [upd_b1[l].reshape(1, HID) for l in range(DEPTH)]

    xp, p2 = _tc_prep(x, w1b[0], nb1[0])
    e = None
    out = None
    for l in range(DEPTH):
        xs, pd = _sc_gather(xp, p2, srcm, dstm)
        if l == 0:
            e, msg = _tc_edge0(xs, pd, ea, e_proj_W, be, w1a[l], w1c[l],
                               upd_W2[l], b2[l])
        else:
            e, msg = _tc_edge(xs, pd, e, w1a[l], w1c[l], upd_W2[l], b2[l])
        aggr = _sc_aggr(msg, dstm, zeros)
        if l < DEPTH - 1:
            x, xp, p2 = _tc_node(x, aggr[0], aggr[1], conv_W1[l], cb1[l],
                             conv_W2[l], cb2[l], gam[l], bet[l],
                             w1b[l + 1], nb1[l + 1])
        else:
            out = _tc_last(x, aggr[0], aggr[1], conv_W1[l], cb1[l],
                           conv_W2[l], cb2[l], gam[l], bet[l], batch_row,
                           ro_W, ro_b.reshape(1, H))
    return out


# speedup vs baseline: 2.6730x; 1.0391x over previous
import functools

import jax
import jax.numpy as jnp
from jax import lax
from jax.experimental import pallas as pl
from jax.experimental.pallas import tpu as pltpu
from jax.experimental.pallas import tpu_sc as plsc

N = 10000
E = 320000
H = 128
EIN = 16
DEPTH = 5
G = 64
HID = 64
BN_EPS = 1e-5

NW = 32
EPAD = 327680
NSPLIT = 2
EPH = EPAD // NSPLIT
EWH = EPH // NW
IDXROWS = EPAD // 128
IRH = EWH // 128
CA = 128
NCH_A = EWH // CA
DA = 4
CC = 128
NCH_C = EWH // CC
NP = 10240
NPR = NP // 16

_mesh = plsc.VectorSubcoreMesh(core_axis_name="c", subcore_axis_name="s")


def _sc_gather_body(x_hbm, p2_hbm, srcm_hbm, dstm_hbm, xs_hbm, pd_hbm,
                    idx_s, idx_d, xs_v, pd_v,
                    sem_i, sem_g0, sem_g1, sem_g2, sem_g3,
                    sem_w0, sem_w1, sem_w2, sem_w3):
    cid = lax.axis_index("c")
    sid = lax.axis_index("s")
    wid = sid * 2 + cid
    sem_g = (sem_g0, sem_g1, sem_g2, sem_g3)
    sem_w = (sem_w0, sem_w1, sem_w2, sem_w3)

    r0 = wid * IRH
    i0 = pltpu.async_copy(srcm_hbm.at[pl.ds(r0, IRH)], idx_s, sem_i)
    i1 = pltpu.async_copy(dstm_hbm.at[pl.ds(r0, IRH)], idx_d, sem_i)
    i0.wait()
    i1.wait()

    LAG = DA - 1
    gd = [None] * NCH_A
    wd = [None] * NCH_A
    for k in range(NCH_A + LAG):
        if k < NCH_A:
            b = k % DA
            if k >= DA:
                for d in wd[k - DA]:
                    d.wait()
            gd[k] = [
                pltpu.async_copy(x_hbm.at[idx_s.at[k]],
                                 xs_v.at[b], sem_g[b]),
                pltpu.async_copy(p2_hbm.at[idx_d.at[k]],
                                 pd_v.at[b], sem_g[b]),
            ]
        if k >= LAG:
            kp = k - LAG
            bp = kp % DA
            for d in gd[kp]:
                d.wait()
            b0 = wid * EWH + kp * CA
            wd[kp] = [
                pltpu.async_copy(xs_v.at[bp], xs_hbm.at[pl.ds(b0, CA)],
                                 sem_w[bp]),
                pltpu.async_copy(pd_v.at[bp], pd_hbm.at[pl.ds(b0, CA)],
                                 sem_w[bp]),
            ]
    for k in range(NCH_A - DA, NCH_A):
        for d in wd[k]:
            d.wait()


_sc_gather = functools.partial(
    pl.kernel,
    out_type=(
        jax.ShapeDtypeStruct((EPH, H // 2), jnp.int32),
        jax.ShapeDtypeStruct((EPH, HID // 2), jnp.int32),
    ),
    mesh=_mesh,
    scratch_types=[
        pltpu.VMEM((IRH, 128), jnp.int32),
        pltpu.VMEM((IRH, 128), jnp.int32),
        pltpu.VMEM((DA, CA, H // 2), jnp.int32),
        pltpu.VMEM((DA, CA, HID // 2), jnp.int32),
        pltpu.SemaphoreType.DMA,
        pltpu.SemaphoreType.DMA,
        pltpu.SemaphoreType.DMA,
        pltpu.SemaphoreType.DMA,
        pltpu.SemaphoreType.DMA,
        pltpu.SemaphoreType.DMA,
        pltpu.SemaphoreType.DMA,
        pltpu.SemaphoreType.DMA,
        pltpu.SemaphoreType.DMA,
    ],
    compiler_params=pltpu.CompilerParams(use_tc_tiling_on_sc=False),
)(_sc_gather_body)


def _sc_aggr_body(msg_hbm, dstm_hbm, zeros_hbm, aggr_hbm,
                  idx_d, msg_v, aggr_sh, sem_m0, sem_m1, sem_i0, sem_i1):
    cid = lax.axis_index("c")
    sid = lax.axis_index("s")
    wid = sid * 2 + cid
    sem_m = (sem_m0, sem_m1)
    sem_i = (sem_i0, sem_i1)

    zd = pltpu.async_copy(zeros_hbm.at[pl.ds(sid * NPR, NPR)],
                          aggr_sh.at[pl.ds(sid * NPR, NPR)], sem_i[0])
    pltpu.sync_copy(dstm_hbm.at[pl.ds(wid * IRH, IRH)], idx_d)
    zd.wait()
    plsc.subcore_barrier()

    md = [None] * NCH_C
    for k in range(NCH_C + 1):
        if k < NCH_C:
            b = k % 2
            b0 = wid * EWH + k * CC
            md[k] = pltpu.async_copy(msg_hbm.at[pl.ds(b0, CC)],
                                     msg_v.at[b], sem_m[b])
        if k >= 1:
            kp = k - 1
            bp = kp % 2
            md[kp].wait()
            pltpu.sync_copy(msg_v.at[bp], aggr_sh.at[idx_d.at[kp]],
                            add=True)

    plsc.subcore_barrier()
    pltpu.sync_copy(aggr_sh.at[pl.ds(sid * NPR, NPR)],
                    aggr_hbm.at[cid, pl.ds(sid * NPR, NPR)])


_sc_aggr = functools.partial(
    pl.kernel,
    out_type=jax.ShapeDtypeStruct((2, NP, H), jnp.float32),
    mesh=_mesh,
    scratch_types=[
        pltpu.VMEM((IRH, 128), jnp.int32),
        pltpu.VMEM((2, CC, H), jnp.float32),
        pltpu.VMEM_SHARED((NP, H), jnp.float32),
        pltpu.SemaphoreType.DMA,
        pltpu.SemaphoreType.DMA,
        pltpu.SemaphoreType.DMA,
        pltpu.SemaphoreType.DMA,
    ],
)(_sc_aggr_body)


EB = 2048


def _dot(a, b):
    return jnp.dot(a, b, preferred_element_type=jnp.float32)


def _rtne(v):
    b = lax.bitcast_convert_type(v, jnp.int32)
    r = b + jnp.int32(0x7FFF) + lax.shift_right_logical(b, 16).astype(jnp.int32) % 2
    return r & (-65536)


def _pack(v, f):
    ra = _rtne(v[:, :f])
    rb = _rtne(v[:, f:])
    return ra | lax.shift_right_logical(rb, 16).astype(jnp.int32)


def _unpack(p):
    hi = lax.bitcast_convert_type(p & (-65536), jnp.float32)
    lo = lax.bitcast_convert_type(lax.shift_left(p, 16), jnp.float32)
    return jnp.concatenate([hi, lo], axis=1)


def _tc_edge_body(xs_ref, pd_ref, e_ref, w1a_ref, w1c_ref, w2_ref, b2_ref,
                  eo_ref, msg_ref):
    e = _unpack(e_ref[...])
    xs = _unpack(xs_ref[...])
    h = jnp.maximum(_unpack(pd_ref[...]) + _dot(xs, w1a_ref[...])
                    + _dot(e, w1c_ref[...]), 0.0)
    eo = e + _dot(h, w2_ref[...]) + b2_ref[...]
    eo_ref[...] = _pack(eo, H // 2)
    msg_ref[...] = jnp.maximum(xs + eo, 0.0)


_tc_edge = pl.pallas_call(
    _tc_edge_body,
    grid=(EPH // EB,),
    in_specs=[
        pl.BlockSpec((EB, H // 2), lambda i: (i, 0)),
        pl.BlockSpec((EB, HID // 2), lambda i: (i, 0)),
        pl.BlockSpec((EB, H // 2), lambda i: (i, 0)),
        pl.BlockSpec((H, HID), lambda i: (0, 0)),
        pl.BlockSpec((H, HID), lambda i: (0, 0)),
        pl.BlockSpec((HID, H), lambda i: (0, 0)),
        pl.BlockSpec((1, H), lambda i: (0, 0)),
    ],
    out_specs=[
        pl.BlockSpec((EB, H // 2), lambda i: (i, 0)),
        pl.BlockSpec((EB, H), lambda i: (i, 0)),
    ],
    out_shape=(
        jax.ShapeDtypeStruct((EPH, H // 2), jnp.int32),
        jax.ShapeDtypeStruct((EPH, H), jnp.float32),
    ),
)


def _tc_edge0_body(xs_ref, pd_ref, ea_ref, we_ref, be_ref, w1a_ref, w1c_ref,
                   w2_ref, b2_ref, eo_ref, msg_ref):
    e = _dot(ea_ref[...], we_ref[...]) + be_ref[...]
    xs = _unpack(xs_ref[...])
    h = jnp.maximum(_unpack(pd_ref[...]) + _dot(xs, w1a_ref[...])
                    + _dot(e, w1c_ref[...]), 0.0)
    eo = e + _dot(h, w2_ref[...]) + b2_ref[...]
    eo_ref[...] = _pack(eo, H // 2)
    msg_ref[...] = jnp.maximum(xs + eo, 0.0)


_tc_edge0 = pl.pallas_call(
    _tc_edge0_body,
    grid=(EPH // EB,),
    in_specs=[
        pl.BlockSpec((EB, H // 2), lambda i: (i, 0)),
        pl.BlockSpec((EB, HID // 2), lambda i: (i, 0)),
        pl.BlockSpec((EB, EIN), lambda i: (i, 0)),
        pl.BlockSpec((EIN, H), lambda i: (0, 0)),
        pl.BlockSpec((1, H), lambda i: (0, 0)),
        pl.BlockSpec((H, HID), lambda i: (0, 0)),
        pl.BlockSpec((H, HID), lambda i: (0, 0)),
        pl.BlockSpec((HID, H), lambda i: (0, 0)),
        pl.BlockSpec((1, H), lambda i: (0, 0)),
    ],
    out_specs=[
        pl.BlockSpec((EB, H // 2), lambda i: (i, 0)),
        pl.BlockSpec((EB, H), lambda i: (i, 0)),
    ],
    out_shape=(
        jax.ShapeDtypeStruct((EPH, H // 2), jnp.int32),
        jax.ShapeDtypeStruct((EPH, H), jnp.float32),
    ),
)


def _node_update(x, a0, a1, w1, b1, w2, b2, gam, bet):
    t = x + (a0[0, 0:N, :] + a0[1, 0:N, :]) + (a1[0, 0:N, :] + a1[1, 0:N, :])
    u = _dot(jnp.maximum(_dot(t, w1) + b1, 0.0), w2) + b2
    mean = jnp.mean(u, axis=0, keepdims=True)
    var = jnp.mean((u - mean) * (u - mean), axis=0, keepdims=True)
    xb = (u - mean) * lax.rsqrt(var + BN_EPS) * gam + bet
    return x + jnp.maximum(xb, 0.0)


def _tc_node_body(x_ref, a0_ref, a1_ref, w1_ref, b1_ref, w2_ref, b2_ref,
                  gam_ref, bet_ref, nwb_ref, nb_ref, xo_ref, xp_ref, p2_ref):
    xn = _node_update(x_ref[...], a0_ref[...], a1_ref[...], w1_ref[...],
                      b1_ref[...], w2_ref[...], b2_ref[...], gam_ref[...],
                      bet_ref[...])
    xo_ref[...] = xn
    xp_ref[...] = _pack(xn, H // 2)
    p2_ref[...] = _pack(_dot(xn, nwb_ref[...]) + nb_ref[...], HID // 2)


_tc_node = pl.pallas_call(
    _tc_node_body,
    out_shape=(
        jax.ShapeDtypeStruct((N, H), jnp.float32),
        jax.ShapeDtypeStruct((N, H // 2), jnp.int32),
        jax.ShapeDtypeStruct((N, HID // 2), jnp.int32),
    ),
)


def _tc_last_body(x_ref, a0_ref, a1_ref, w1_ref, b1_ref, w2_ref, b2_ref,
                  gam_ref, bet_ref, batch_ref, row_ref, rob_ref, out_ref):
    xn = _node_update(x_ref[...], a0_ref[...], a1_ref[...], w1_ref[...],
                      b1_ref[...], w2_ref[...], b2_ref[...], gam_ref[...],
                      bet_ref[...])
    oh = (lax.broadcasted_iota(jnp.int32, (G, 1), 0)
          == batch_ref[...]).astype(jnp.float32)
    sums = _dot(oh, xn)
    cnt = jnp.sum(oh, axis=1, keepdims=True)
    g = sums / jnp.maximum(cnt, 1.0)
    out_ref[...] = jnp.maximum(_dot(g, row_ref[...]) + rob_ref[...], 0.0)


_tc_last = pl.pallas_call(
    _tc_last_body,
    out_shape=jax.ShapeDtypeStruct((G, H), jnp.float32),
)


def _tc_prep_body(x_ref, nwb_ref, nb_ref, xp_ref, p2_ref):
    xp_ref[...] = _pack(x_ref[...], H // 2)
    p2_ref[...] = _pack(_dot(x_ref[...], nwb_ref[...]) + nb_ref[...], HID // 2)


_tc_prep = pl.pallas_call(
    _tc_prep_body,
    out_shape=(
        jax.ShapeDtypeStruct((N, H // 2), jnp.int32),
        jax.ShapeDtypeStruct((N, HID // 2), jnp.int32),
    ),
)


def kernel(x, edge_index, edge_attr, batch, e_proj_W, e_proj_b, upd_W1,
           upd_b1, upd_W2, upd_b2, conv_W1, conv_b1, conv_W2, conv_b2,
           bn_gamma, bn_beta, ro_W, ro_b):
    pad = EPAD - E
    src = jnp.concatenate([edge_index[0], jnp.zeros((pad,), jnp.int32)])
    dst = jnp.concatenate([edge_index[1], jnp.full((pad,), N, jnp.int32)])
    srcm = src.reshape(NSPLIT, IDXROWS // NSPLIT, 128)
    dstm = dst.reshape(NSPLIT, IDXROWS // NSPLIT, 128)
    ea3 = jnp.concatenate([edge_attr, jnp.zeros((pad, EIN), jnp.float32)]
                          ).reshape(NSPLIT, EPH, EIN)
    zeros = jnp.zeros((NP, H), jnp.float32)
    batch_row = batch.reshape(1, N)

    be = e_proj_b.reshape(1, H)
    b2 = [upd_b2[l].reshape(1, H) for l in range(DEPTH)]
    cb1 = [conv_b1[l].reshape(1, H) for l in range(DEPTH)]
    cb2 = [conv_b2[l].reshape(1, H) for l in range(DEPTH)]
    gam = [bn_gamma[l].reshape(1, H) for l in range(DEPTH)]
    bet = [bn_beta[l].reshape(1, H) for l in range(DEPTH)]
    w1a = [upd_W1[l, :H, :] for l in range(DEPTH)]
    w1b = [upd_W1[l, H:2 * H, :] for l in range(DEPTH)]
    w1c = [upd_W1[l, 2 * H:, :] for l in range(DEPTH)]
    nb1 = [upd_b1[l].reshape(1, HID) for l in range(DEPTH)]

    xp, p2 = _tc_prep(x, w1b[0], nb1[0])
    e = [None, None]
    out = None
    for l in range(DEPTH):
        msg = [None, None]
        aggr = [None, None]
        for h in range(NSPLIT):
            xs, pd = _sc_gather(xp, p2, srcm[h], dstm[h])
            if l == 0:
                e[h], msg[h] = _tc_edge0(xs, pd, ea3[h], e_proj_W, be,
                                         w1a[l], w1c[l], upd_W2[l], b2[l])
            else:
                e[h], msg[h] = _tc_edge(xs, pd, e[h], w1a[l], w1c[l],
                                        upd_W2[l], b2[l])
            aggr[h] = _sc_aggr(msg[h], dstm[h], zeros)
        if l < DEPTH - 1:
            x, xp, p2 = _tc_node(x, aggr[0], aggr[1], conv_W1[l], cb1[l],
                                 conv_W2[l], cb2[l], gam[l], bet[l],
                                 w1b[l + 1], nb1[l + 1])
        else:
            out = _tc_last(x, aggr[0], aggr[1], conv_W1[l], cb1[l],
                           conv_W2[l], cb2[l], gam[l], bet[l], batch_row,
                           ro_W, ro_b.reshape(1, H))
    return out
